# baseline (speedup 1.0000x reference)
import functools

import jax
import jax.numpy as jnp
from jax import lax
from jax.experimental import pallas as pl
from jax.experimental.pallas import tpu as pltpu
from jax.experimental.pallas import tpu_sc as plsc

B, S, D = 2, 2048, 768
E = 16
TOP_K = 2
DH = 2048
THRESHOLD = 0.2
CAP_FACTOR = 1.25
BALANCE_COEF = 0.01
ZLOSS_COEF = 0.001
N = B * S
R2 = 2 * N
C = int(TOP_K * N / E * CAP_FACTOR)
NSLOT = E * C
DUMP = NSLOT
NROWS = NSLOT + 8

CH = 512
NCH = R2 // CH

NC, NS, LANES = 2, 16, 16
NW = NC * NS
DISP_PER_W = R2 // NW
COMB_PER_W = N // NW
SUB = 64


def _router_body(x_ref, wg_ref, sd_ref, sc_ref, g_ref, aux_ref, bal_ref,
                 z_ref, oh_scr, cs_scr):
    xt = x_ref[...]
    wg = wg_ref[...]
    logits = jnp.dot(xt, wg, preferred_element_type=jnp.float32)

    m = jnp.max(logits, axis=1, keepdims=True)
    ex = jnp.exp(logits - m)
    se = jnp.sum(ex, axis=1, keepdims=True)
    probs = ex / se
    lse = jnp.log(se[:, 0]) + m[:, 0]
    zloss = jnp.mean(lse * lse)

    idx16 = lax.broadcasted_iota(jnp.int32, (N, E), 1)
    m1 = jnp.max(probs, axis=1, keepdims=True)
    a1 = jnp.min(jnp.where(probs == m1, idx16, E), axis=1)
    masked = jnp.where(idx16 == a1[:, None], -jnp.inf, probs)
    m2 = jnp.max(masked, axis=1, keepdims=True)
    a2 = jnp.min(jnp.where(masked == m2, idx16, E), axis=1)

    g1 = m1[:, 0]
    g2 = m2[:, 0]
    den = g1 + g2 + 1e-9
    keep2 = g2 >= THRESHOLD
    gn0 = g1 / den
    gn1 = jnp.where(keep2, g2 / den, 0.0)

    oh1 = (idx16 == a1[:, None]).astype(jnp.float32)
    fe = jnp.mean(oh1, axis=0)
    me = jnp.mean(probs, axis=0)
    bal = E * jnp.sum(fe * me)

    aux_ref[...] = jnp.full((1, 1), BALANCE_COEF * bal + ZLOSS_COEF * zloss)
    bal_ref[...] = jnp.full((1, 1), bal)
    z_ref[...] = jnp.full((1, 1), zloss)

    oh2 = (idx16 == a2[:, None]).astype(jnp.float32) * keep2[:, None]
    oh_scr[...] = jnp.concatenate([oh1, oh2], axis=0)

    r = lax.broadcasted_iota(jnp.int32, (CH, CH), 0)
    c = lax.broadcasted_iota(jnp.int32, (CH, CH), 1)
    ltri = (r > c).astype(jnp.float32)

    def chunk(i, carry):
        blk = oh_scr[pl.ds(i * CH, CH), :]
        excl = jnp.dot(ltri, blk, preferred_element_type=jnp.float32) + carry
        cs_scr[pl.ds(i * CH, CH), :] = excl
        return carry + jnp.sum(blk, axis=0, keepdims=True)

    lax.fori_loop(0, NCH, chunk, jnp.zeros((1, E), jnp.float32))

    oh = oh_scr[...]
    pos = jnp.sum(oh * cs_scr[...], axis=1)
    valid = jnp.sum(oh, axis=1) > 0.0
    within = valid & (pos < C)
    eflat = jnp.concatenate([a1, a2], axis=0)
    slot = eflat * C + pos.astype(jnp.int32)
    sd_ref[...] = jnp.where(within, slot, DUMP)
    safe = jnp.min(jnp.where(within, slot, NSLOT))
    sc_ref[...] = jnp.where(within, slot, safe)
    gflat = jnp.concatenate([gn0, gn1], axis=0) * within.astype(jnp.float32)
    g_ref[...] = jnp.broadcast_to(gflat[:, None], (R2, LANES))


def _router(xt, Wg):
    return pl.pallas_call(
        _router_body,
        out_shape=(
            jax.ShapeDtypeStruct((R2,), jnp.int32),
            jax.ShapeDtypeStruct((R2,), jnp.int32),
            jax.ShapeDtypeStruct((R2, LANES), jnp.float32),
            jax.ShapeDtypeStruct((1, 1), jnp.float32),
            jax.ShapeDtypeStruct((1, 1), jnp.float32),
            jax.ShapeDtypeStruct((1, 1), jnp.float32),
        ),
        scratch_shapes=[
            pltpu.VMEM((R2, E), jnp.float32),
            pltpu.VMEM((R2, E), jnp.float32),
        ],
    )(xt, Wg)


def _dispatch_body(x_hbm, sd_hbm, disp_hbm, idx_v, buf_v, sem):
    wid = lax.axis_index("s") * NC + lax.axis_index("c")
    row_base = wid * DISP_PER_W
    tok_base = lax.rem(row_base, N)
    for ch in range(DISP_PER_W // SUB):
        pltpu.sync_copy(x_hbm.at[pl.ds(tok_base + ch * SUB, SUB)], buf_v)
        pltpu.sync_copy(sd_hbm.at[pl.ds(row_base + ch * SUB, SUB)], idx_v)
        pltpu.async_copy(buf_v, disp_hbm.at[idx_v], sem).wait()


def _dispatch(xt, sd):
    mesh = plsc.VectorSubcoreMesh(core_axis_name="c", subcore_axis_name="s")
    kfn = pl.kernel(
        _dispatch_body,
        mesh=mesh,
        out_type=jax.ShapeDtypeStruct((NROWS, D), jnp.float32),
        scratch_types=[
            pltpu.VMEM((SUB,), jnp.int32),
            pltpu.VMEM((SUB, D), jnp.float32),
            pltpu.SemaphoreType.DMA,
        ],
    )
    return kfn(xt, sd)


BC = 128
NBC = C // BC


def _ffn_body(d_ref, lg_ref, lb_ref, w1_ref, b1_ref, w2_ref, b2_ref, o_ref):
    xb = d_ref[...]
    mu = jnp.mean(xb, axis=1, keepdims=True)
    xc = xb - mu
    var = jnp.mean(xc * xc, axis=1, keepdims=True)
    h = xc * lax.rsqrt(var + 1e-5) * lg_ref[0] + lb_ref[0]
    h1 = jnp.dot(h, w1_ref[0], preferred_element_type=jnp.float32) + b1_ref[0]
    h1 = jnp.where(h1 > 0, h1, 0.01 * h1)
    o_ref[...] = jnp.dot(h1, w2_ref[0], preferred_element_type=jnp.float32) \
        + b2_ref[0]


def _ffn(disp, ln_g, ln_b, W1, b1, W2, b2):
    return pl.pallas_call(
        _ffn_body,
        grid=(E, NBC),
        in_specs=[
            pl.BlockSpec((BC, D), lambda e, c: (e * NBC + c, 0)),
            pl.BlockSpec((1, 1, D), lambda e, c: (e, 0, 0)),
            pl.BlockSpec((1, 1, D), lambda e, c: (e, 0, 0)),
            pl.BlockSpec((1, D, DH), lambda e, c: (e, 0, 0)),
            pl.BlockSpec((1, 1, DH), lambda e, c: (e, 0, 0)),
            pl.BlockSpec((1, DH, D), lambda e, c: (e, 0, 0)),
            pl.BlockSpec((1, 1, D), lambda e, c: (e, 0, 0)),
        ],
        out_specs=pl.BlockSpec((BC, D), lambda e, c: (e * NBC + c, 0)),
        out_shape=jax.ShapeDtypeStruct((NSLOT, D), jnp.float32),
        compiler_params=pltpu.CompilerParams(
            dimension_semantics=("arbitrary", "arbitrary")),
    )(disp, ln_g.reshape(E, 1, D), ln_b.reshape(E, 1, D), W1,
      b1.reshape(E, 1, DH), W2, b2.reshape(E, 1, D))


def _combine_body(h_hbm, sc_hbm, g_hbm, out_hbm,
                  idx0_v, idx1_v, g0_v, g1_v, buf0_v, buf1_v, sem):
    wid = lax.axis_index("s") * NC + lax.axis_index("c")
    for ch in range(COMB_PER_W // SUB):
        base = wid * COMB_PER_W + ch * SUB
        pltpu.sync_copy(sc_hbm.at[pl.ds(base, SUB)], idx0_v)
        pltpu.sync_copy(sc_hbm.at[pl.ds(N + base, SUB)], idx1_v)
        pltpu.sync_copy(g_hbm.at[pl.ds(base, SUB)], g0_v)
        pltpu.sync_copy(g_hbm.at[pl.ds(N + base, SUB)], g1_v)
        pltpu.async_copy(h_hbm.at[idx0_v], buf0_v, sem).wait()
        pltpu.async_copy(h_hbm.at[idx1_v], buf1_v, sem).wait()

        def row(r, _):
            gr0 = g0_v[r, :]
            gr1 = g1_v[r, :]
            for dd in range(D // LANES):
                a = buf0_v[r, pl.ds(dd * LANES, LANES)]
                b = buf1_v[r, pl.ds(dd * LANES, LANES)]
                buf0_v[r, pl.ds(dd * LANES, LANES)] = a * gr0 + b * gr1
            return 0

        lax.fori_loop(0, SUB, row, 0)
        pltpu.sync_copy(buf0_v, out_hbm.at[pl.ds(base, SUB)])


def _combine(h, sc, g):
    mesh = plsc.VectorSubcoreMesh(core_axis_name="c", subcore_axis_name="s")
    kfn = pl.kernel(
        _combine_body,
        mesh=mesh,
        out_type=jax.ShapeDtypeStruct((N, D), jnp.float32),
        scratch_types=[
            pltpu.VMEM((SUB,), jnp.int32),
            pltpu.VMEM((SUB,), jnp.int32),
            pltpu.VMEM((SUB, LANES), jnp.float32),
            pltpu.VMEM((SUB, LANES), jnp.float32),
            pltpu.VMEM((SUB, D), jnp.float32),
            pltpu.VMEM((SUB, D), jnp.float32),
            pltpu.SemaphoreType.DMA,
        ],
    )
    return kfn(h, sc, g)


def kernel(x, Wg, ln_g, ln_b, W1, b1, W2, b2):
    xt = x.reshape(N, D)
    sd, sc, g, aux, bal, z = _router(xt, Wg)
    disp = _dispatch(xt, sd)
    h = _ffn(disp, ln_g, ln_b, W1, b1, W2, b2)
    out = _combine(h, sc, g)
    return (out.reshape(B, S, D), aux[0, 0], bal[0, 0], z[0, 0])
